# Initial kernel scaffold; baseline (speedup 1.0000x reference)
import jax
import jax.numpy as jnp
from jax.experimental import pallas as pl


def kernel(node_features, edge_indices, mW1, mb1, mW2, mb2, uW1, ub1, uW2, ub2):
    raise NotImplementedError("write your pallas kernel here")


# speedup vs baseline: 1.6030x; 1.6030x over previous
import jax
import jax.numpy as jnp
from jax import lax
from jax.experimental import pallas as pl
from jax.experimental.pallas import tpu as pltpu
from jax.experimental.pallas import tpu_sc as plsc

N = 10000
E = 320000
H = 128
NC, NS = 2, 16
NW = NC * NS
CH = 80
NREAL = E // (NS * CH)
NCHUNK = 256
NPH = 4
PCH = NCHUNK // NPH
NHALF = 5120
NPAD = 2 * NHALF
RPW = NHALF // NS
RCH = 80
VB = H // 16
ROWBLK = 1000


def _prep_body(x_ref, w1_ref, b1_ref, a_ref, b_ref):
    x = x_ref[...]
    dn = (((1,), (1,)), ((), ()))
    a_ref[...] = lax.dot_general(x, w1_ref[:, :H], dn,
                                 precision=lax.Precision.HIGHEST,
                                 preferred_element_type=jnp.float32)
    b_ref[...] = lax.dot_general(x, w1_ref[:, H:], dn,
                                 precision=lax.Precision.HIGHEST,
                                 preferred_element_type=jnp.float32) + b1_ref[...]


def _tc_pre(x, mW1, mb1):
    return pl.pallas_call(
        _prep_body,
        grid=(N // ROWBLK,),
        in_specs=[
            pl.BlockSpec((ROWBLK, H), lambda i: (i, 0)),
            pl.BlockSpec((H, 2 * H), lambda i: (0, 0)),
            pl.BlockSpec((1, H), lambda i: (0, 0)),
        ],
        out_specs=[
            pl.BlockSpec((ROWBLK, H), lambda i: (i, 0)),
            pl.BlockSpec((ROWBLK, H), lambda i: (i, 0)),
        ],
        out_shape=[
            jax.ShapeDtypeStruct((N, H), jnp.float32),
            jax.ShapeDtypeStruct((N, H), jnp.float32),
        ],
    )(x, mW1, mb1.reshape(1, H))


def _sc_body(a_hbm, b_hbm, src_hbm, dst_hbm, out_hbm, cnt_hbm,
             sidx, didx, abuf0, bbuf0, mbuf0,
             zbuf, cntbuf, acc,
             sga0, ssc0):
    c = lax.axis_index("c")
    s = lax.axis_index("s")
    wid = c * NS + s
    base = c * NHALF
    abuf = [abuf0]
    bbuf = [bbuf0]
    mbuf = [mbuf0]
    sga = [sga0]
    sgb = [sga0]
    ssc = [ssc0]


    def zcnt(r, _):
        cntbuf[pl.ds(r * 16, 16)] = jnp.zeros((16,), jnp.float32)
        return 0
    lax.fori_loop(0, NPAD // 16, zcnt, 0)

    def zrow(r, _):
        for j in range(VB):
            zbuf[r, pl.ds(j * 16, 16)] = jnp.zeros((16,), jnp.float32)
        return 0
    lax.fori_loop(0, RCH, zrow, 0)
    for k in range(RPW // RCH):
        pltpu.sync_copy(zbuf, acc.at[pl.ds(s * RPW + k * RCH, RCH)])
    plsc.subcore_barrier()

    def issue_gather(i, b):
        pltpu.async_copy(a_hbm.at[sidx.at[i]], abuf[b], sga[b])
        pltpu.async_copy(b_hbm.at[didx.at[i]], bbuf[b], sgb[b])

    def wait_gather(i, b):
        pltpu.make_async_copy(a_hbm.at[sidx.at[i]], abuf[b], sga[b]).wait()
        pltpu.make_async_copy(b_hbm.at[didx.at[i]], bbuf[b], sgb[b]).wait()

    def issue_scatter(i, b):
        pltpu.async_copy(mbuf[b], acc.at[didx.at[i]], ssc[b], add=True)

    def wait_scatter(i, b):
        pltpu.make_async_copy(mbuf[b], acc.at[didx.at[i]], ssc[b]).wait()

    lanes = lax.iota(jnp.int32, 16)
    cscale = jnp.where(c == 0, 1.0, 0.0)

    def compute(i, b, validf):
        def cgrp(g, _):
            gsl = pl.ds(g * 16, 16)
            dv = didx[i, gsl]
            lv = dv - base
            ok = jnp.logical_and(lv >= 0, lv < NHALF)
            okf = jnp.where(ok, validf, 0.0)
            incs = cscale * validf
            didx[i, gsl] = jnp.where(ok, lv, 0)
            for l in range(16):
                d = dv[l]
                albase = jnp.bitwise_and(d, -16)
                incv = jnp.where(lanes == d - albase, incs, 0.0)
                cw = pl.ds(albase, 16)
                cntbuf[cw] = cntbuf[cw] + incv
                r = g * 16 + l
                for j in range(VB):
                    sl = pl.ds(j * 16, 16)
                    mbuf[b][r, sl] = jnp.maximum(
                        abuf[b][r, sl] + bbuf[b][r, sl], 0.0) * okf[l]
            return 0
        lax.fori_loop(0, CH // 16, cgrp, 0)

    for p in range(NPH):
        def seq(i, _):
            validf = jnp.where(p * PCH + i < NREAL, 1.0, 0.0)
            issue_gather(i, 0)
            wait_gather(i, 0)
            compute(i, 0, validf)
            issue_scatter(i, 0)
            wait_scatter(i, 0)
            return 0

        pltpu.sync_copy(src_hbm.at[s, pl.ds(p * PCH, PCH)], sidx)
        pltpu.sync_copy(dst_hbm.at[s, pl.ds(p * PCH, PCH)], didx)
        lax.fori_loop(0, PCH, seq, 0)

    pltpu.sync_copy(cntbuf, cnt_hbm.at[pl.ds(wid * NPAD, NPAD)])

    plsc.subcore_barrier()
    for k in range(RPW // RCH):
        row = s * RPW + k * RCH
        pltpu.sync_copy(acc.at[pl.ds(row, RCH)], zbuf)
        pltpu.sync_copy(zbuf, out_hbm.at[pl.ds(c * NHALF + row, RCH)])


def _sc_segsum(a, b, src_r, dst_r):
    mesh = plsc.VectorSubcoreMesh(core_axis_name="c", subcore_axis_name="s")
    f = pl.kernel(
        _sc_body,
        out_type=[
            jax.ShapeDtypeStruct((NPAD, H), jnp.float32),
            jax.ShapeDtypeStruct((NW * NPAD,), jnp.float32),
        ],
        mesh=mesh,
        scratch_types=[
            pltpu.VMEM((PCH, CH), jnp.int32),
            pltpu.VMEM((PCH, CH), jnp.int32),
            pltpu.VMEM((CH, H), jnp.float32),
            pltpu.VMEM((CH, H), jnp.float32),
            pltpu.VMEM((CH, H), jnp.float32),
            pltpu.VMEM((RCH, H), jnp.float32),
            pltpu.VMEM((NPAD,), jnp.float32),
            pltpu.MemorySpace.VMEM_SHARED((NHALF, H), jnp.float32),
            pltpu.SemaphoreType.DMA,
            pltpu.SemaphoreType.DMA,
        ],
    )
    return f(a, b, src_r, dst_r)


def _post_body(p_ref, cnt_ref, x_ref, w2_ref, b2_ref, uw1_ref,
               ub1_ref, uw2_ref, ub2_ref, out_ref):
    S = p_ref[...]
    dn = (((1,), (1,)), ((), ()))
    messages = lax.dot_general(S, w2_ref[...], dn,
                               precision=lax.Precision.HIGHEST,
                               preferred_element_type=jnp.float32)
    messages = messages + cnt_ref[...] * b2_ref[...]
    x = x_ref[...]
    h2 = lax.dot_general(x, uw1_ref[:, :H], dn,
                         precision=lax.Precision.HIGHEST,
                         preferred_element_type=jnp.float32)
    h2 = h2 + lax.dot_general(messages, uw1_ref[:, H:], dn,
                              precision=lax.Precision.HIGHEST,
                              preferred_element_type=jnp.float32)
    h2 = jnp.maximum(h2 + ub1_ref[...], 0.0)
    out = lax.dot_general(h2, uw2_ref[...], dn,
                          precision=lax.Precision.HIGHEST,
                          preferred_element_type=jnp.float32)
    out_ref[...] = out + ub2_ref[...]


def _tc_post(parts, cnts, x, mW2, mb2, uW1, ub1, uW2, ub2):
    p = parts[:N]
    cnt_col = cnts.reshape(NW, NPAD).sum(axis=0)[:N].reshape(N, 1)
    return pl.pallas_call(
        _post_body,
        grid=(N // ROWBLK,),
        in_specs=[
            pl.BlockSpec((ROWBLK, H), lambda i: (i, 0)),
            pl.BlockSpec((ROWBLK, 1), lambda i: (i, 0)),
            pl.BlockSpec((ROWBLK, H), lambda i: (i, 0)),
            pl.BlockSpec((H, H), lambda i: (0, 0)),
            pl.BlockSpec((1, H), lambda i: (0, 0)),
            pl.BlockSpec((H, 2 * H), lambda i: (0, 0)),
            pl.BlockSpec((1, H), lambda i: (0, 0)),
            pl.BlockSpec((H, H), lambda i: (0, 0)),
            pl.BlockSpec((1, H), lambda i: (0, 0)),
        ],
        out_specs=pl.BlockSpec((ROWBLK, H), lambda i: (i, 0)),
        out_shape=jax.ShapeDtypeStruct((N, H), jnp.float32),
    )(p, cnt_col, x, mW2, mb2.reshape(1, H), uW1, ub1.reshape(1, H),
      uW2, ub2.reshape(1, H))


def kernel(node_features, edge_indices, mW1, mb1, mW2, mb2,
           uW1, ub1, uW2, ub2):
    x = node_features
    ei = edge_indices.astype(jnp.int32)
    pad = jnp.zeros((NS, NCHUNK - NREAL, CH), jnp.int32)
    src_r = jnp.concatenate([ei[:, 0].reshape(NS, NREAL, CH), pad], axis=1)
    dst_r = jnp.concatenate([ei[:, 1].reshape(NS, NREAL, CH), pad], axis=1)
    a, b = _tc_pre(x, mW1, mb1)
    parts, cnts = _sc_segsum(a, b, src_r, dst_r)
    return _tc_post(parts, cnts, x, mW2, mb2, uW1, ub1, uW2, ub2)


# speedup vs baseline: 1.6158x; 1.0079x over previous
import jax
import jax.numpy as jnp
from jax import lax
from jax.experimental import pallas as pl
from jax.experimental.pallas import tpu as pltpu
from jax.experimental.pallas import tpu_sc as plsc

N = 10000
E = 320000
H = 128
NC, NS = 2, 16
NW = NC * NS
CH = 128
EPT = E // NS
NCHUNK = 160
NPH = 4
PCH = NCHUNK // NPH
NHALF = 5120
NPAD = 2 * NHALF
SENT = 2 * NHALF
BROWS = 10248
RPW = NHALF // NS
RCH = 80
VB = H // 16
ROWBLK = 1000


def _prep_body(x_ref, w1_ref, b1_ref, a_ref, b_ref):
    x = x_ref[...]
    dn = (((1,), (1,)), ((), ()))
    a_ref[...] = lax.dot_general(x, w1_ref[:, :H], dn,
                                 precision=lax.Precision.HIGHEST,
                                 preferred_element_type=jnp.float32)
    b_ref[...] = lax.dot_general(x, w1_ref[:, H:], dn,
                                 precision=lax.Precision.HIGHEST,
                                 preferred_element_type=jnp.float32) + b1_ref[...]


def _tc_pre(x, mW1, mb1):
    return pl.pallas_call(
        _prep_body,
        grid=(N // ROWBLK,),
        in_specs=[
            pl.BlockSpec((ROWBLK, H), lambda i: (i, 0)),
            pl.BlockSpec((H, 2 * H), lambda i: (0, 0)),
            pl.BlockSpec((1, H), lambda i: (0, 0)),
        ],
        out_specs=[
            pl.BlockSpec((ROWBLK, H), lambda i: (i, 0)),
            pl.BlockSpec((ROWBLK, H), lambda i: (i, 0)),
        ],
        out_shape=[
            jax.ShapeDtypeStruct((N, H), jnp.float32),
            jax.ShapeDtypeStruct((N, H), jnp.float32),
        ],
    )(x, mW1, mb1.reshape(1, H))


def _sc_body(a_hbm, b_hbm, src_hbm, dst_hbm, out_hbm, cnt_hbm,
             sidx, didx, abuf0, bbuf0, mbuf0,
             zbuf, cntbuf, acc,
             sga0, ssc0):
    c = lax.axis_index("c")
    s = lax.axis_index("s")
    wid = c * NS + s
    base = c * NHALF
    abuf = [abuf0]
    bbuf = [bbuf0]
    mbuf = [mbuf0]
    sga = [sga0]
    sgb = [sga0]
    ssc = [ssc0]


    def zcnt(r, _):
        cntbuf[pl.ds(r * 16, 16)] = jnp.zeros((16,), jnp.float32)
        return 0
    lax.fori_loop(0, (NPAD + 16) // 16, zcnt, 0)

    def zrow(r, _):
        for j in range(VB):
            zbuf[r, pl.ds(j * 16, 16)] = jnp.zeros((16,), jnp.float32)
        return 0
    lax.fori_loop(0, RCH, zrow, 0)
    for k in range(RPW // RCH):
        pltpu.sync_copy(zbuf, acc.at[pl.ds(s * RPW + k * RCH, RCH)])
    plsc.subcore_barrier()

    def issue_gather(i, b):
        pltpu.async_copy(a_hbm.at[sidx.at[i]], abuf[b], sga[b])
        pltpu.async_copy(b_hbm.at[didx.at[i]], bbuf[b], sgb[b])

    def wait_gather(i, b):
        pltpu.make_async_copy(a_hbm.at[sidx.at[i]], abuf[b], sga[b]).wait()
        pltpu.make_async_copy(b_hbm.at[didx.at[i]], bbuf[b], sgb[b]).wait()

    def issue_scatter(i, b):
        pltpu.async_copy(mbuf[b], acc.at[didx.at[i]], ssc[b], add=True)

    def wait_scatter(i, b):
        pltpu.make_async_copy(mbuf[b], acc.at[didx.at[i]], ssc[b]).wait()

    lanes = lax.iota(jnp.int32, 16)
    cscale = jnp.where(c == 0, 1.0, 0.0)

    def compute(i, b):
        def cgrp(g, _):
            gsl = pl.ds(g * 16, 16)
            dv = didx[i, gsl]
            lv = dv - base
            ok = jnp.logical_and(lv >= 0, lv < NHALF)
            okf = jnp.where(ok, 1.0, 0.0)
            didx[i, gsl] = jnp.where(ok, lv, 0)
            for l in range(16):
                d = dv[l]
                albase = jnp.bitwise_and(d, -16)
                incv = jnp.where(lanes == d - albase, cscale, 0.0)
                cw = pl.ds(albase, 16)
                cntbuf[cw] = cntbuf[cw] + incv
                r = g * 16 + l
                for j in range(VB):
                    sl = pl.ds(j * 16, 16)
                    mbuf[b][r, sl] = jnp.maximum(
                        abuf[b][r, sl] + bbuf[b][r, sl], 0.0) * okf[l]
            return 0
        lax.fori_loop(0, CH // 16, cgrp, 0)

    def seq(i, _):
        issue_gather(i, 0)
        wait_gather(i, 0)
        compute(i, 0)
        issue_scatter(i, 0)
        wait_scatter(i, 0)
        return 0

    for p in range(NPH):
        pltpu.sync_copy(src_hbm.at[s, pl.ds(p * PCH, PCH)], sidx)
        pltpu.sync_copy(dst_hbm.at[s, pl.ds(p * PCH, PCH)], didx)
        lax.fori_loop(0, PCH, seq, 0)

    pltpu.sync_copy(cntbuf, cnt_hbm.at[pl.ds(wid * (NPAD + 16), NPAD + 16)])

    plsc.subcore_barrier()
    for k in range(RPW // RCH):
        row = s * RPW + k * RCH
        pltpu.sync_copy(acc.at[pl.ds(row, RCH)], zbuf)
        pltpu.sync_copy(zbuf, out_hbm.at[pl.ds(c * NHALF + row, RCH)])


def _sc_segsum(a, b, src_r, dst_r):
    mesh = plsc.VectorSubcoreMesh(core_axis_name="c", subcore_axis_name="s")
    f = pl.kernel(
        _sc_body,
        out_type=[
            jax.ShapeDtypeStruct((NPAD, H), jnp.float32),
            jax.ShapeDtypeStruct((NW * (NPAD + 16),), jnp.float32),
        ],
        mesh=mesh,
        scratch_types=[
            pltpu.VMEM((PCH, CH), jnp.int32),
            pltpu.VMEM((PCH, CH), jnp.int32),
            pltpu.VMEM((CH, H), jnp.float32),
            pltpu.VMEM((CH, H), jnp.float32),
            pltpu.VMEM((CH, H), jnp.float32),
            pltpu.VMEM((RCH, H), jnp.float32),
            pltpu.VMEM((NPAD + 16,), jnp.float32),
            pltpu.MemorySpace.VMEM_SHARED((NHALF, H), jnp.float32),
            pltpu.SemaphoreType.DMA,
            pltpu.SemaphoreType.DMA,
        ],
    )
    return f(a, b, src_r, dst_r)


def _post_body(p_ref, cnt_ref, x_ref, w2_ref, b2_ref, uw1_ref,
               ub1_ref, uw2_ref, ub2_ref, out_ref):
    S = p_ref[...]
    dn = (((1,), (1,)), ((), ()))
    messages = lax.dot_general(S, w2_ref[...], dn,
                               precision=lax.Precision.HIGHEST,
                               preferred_element_type=jnp.float32)
    messages = messages + cnt_ref[...] * b2_ref[...]
    x = x_ref[...]
    h2 = lax.dot_general(x, uw1_ref[:, :H], dn,
                         precision=lax.Precision.HIGHEST,
                         preferred_element_type=jnp.float32)
    h2 = h2 + lax.dot_general(messages, uw1_ref[:, H:], dn,
                              precision=lax.Precision.HIGHEST,
                              preferred_element_type=jnp.float32)
    h2 = jnp.maximum(h2 + ub1_ref[...], 0.0)
    out = lax.dot_general(h2, uw2_ref[...], dn,
                          precision=lax.Precision.HIGHEST,
                          preferred_element_type=jnp.float32)
    out_ref[...] = out + ub2_ref[...]


def _tc_post(parts, cnts, x, mW2, mb2, uW1, ub1, uW2, ub2):
    p = parts[:N]
    cnt_col = cnts.reshape(NW, NPAD + 16).sum(axis=0)[:N].reshape(N, 1)
    return pl.pallas_call(
        _post_body,
        grid=(N // ROWBLK,),
        in_specs=[
            pl.BlockSpec((ROWBLK, H), lambda i: (i, 0)),
            pl.BlockSpec((ROWBLK, 1), lambda i: (i, 0)),
            pl.BlockSpec((ROWBLK, H), lambda i: (i, 0)),
            pl.BlockSpec((H, H), lambda i: (0, 0)),
            pl.BlockSpec((1, H), lambda i: (0, 0)),
            pl.BlockSpec((H, 2 * H), lambda i: (0, 0)),
            pl.BlockSpec((1, H), lambda i: (0, 0)),
            pl.BlockSpec((H, H), lambda i: (0, 0)),
            pl.BlockSpec((1, H), lambda i: (0, 0)),
        ],
        out_specs=pl.BlockSpec((ROWBLK, H), lambda i: (i, 0)),
        out_shape=jax.ShapeDtypeStruct((N, H), jnp.float32),
    )(p, cnt_col, x, mW2, mb2.reshape(1, H), uW1, ub1.reshape(1, H),
      uW2, ub2.reshape(1, H))


def kernel(node_features, edge_indices, mW1, mb1, mW2, mb2,
           uW1, ub1, uW2, ub2):
    x = node_features
    ei = edge_indices.astype(jnp.int32)
    npad_e = NCHUNK * CH - EPT
    spad = jnp.zeros((NS, npad_e), jnp.int32)
    dpad = jnp.full((NS, npad_e), SENT, jnp.int32)
    src_r = jnp.concatenate([ei[:, 0].reshape(NS, EPT), spad],
                            axis=1).reshape(NS, NCHUNK, CH)
    dst_r = jnp.concatenate([ei[:, 1].reshape(NS, EPT), dpad],
                            axis=1).reshape(NS, NCHUNK, CH)
    a, b = _tc_pre(x, mW1, mb1)
    b = jnp.concatenate([b, jnp.zeros((BROWS - N, H), jnp.float32)], axis=0)
    parts, cnts = _sc_segsum(a, b, src_r, dst_r)
    return _tc_post(parts, cnts, x, mW2, mb2, uW1, ub1, uW2, ub2)
